# baseline (speedup 1.0000x reference)
import functools

import jax
import jax.numpy as jnp
from jax import lax
from jax.experimental import pallas as pl
from jax.experimental.pallas import tpu as pltpu
from jax.experimental.pallas import tpu_sc as plsc

_B = 4096
_D = 64
_NE = 3
_NC, _NS = 2, 16
_NW = _NC * _NS
_BPW = _B // _NW
_L = 16
_IPW = _BPW * _NE
_RG = _IPW // 8


def _body(xf, tbl3, out, iv, rows_v, q, ov, sem):
    wid = lax.axis_index("s") * _NC + lax.axis_index("c")
    base = wid * _BPW
    pltpu.sync_copy(xf.at[pl.ds(wid * _IPW, _IPW)], iv)

    @plsc.parallel_loop(0, _IPW // _L, unroll=2)
    def fire(g):
        vec = iv[pl.ds(g * _L, _L)]
        tvec = lax.shift_right_logical(vec, 3)
        rvec = lax.bitwise_and(vec, 7)
        for e in range(_L):
            pltpu.async_copy(
                tbl3.at[tvec[e], rvec[e]],
                rows_v.at[2 * g + e // 8, e % 8],
                sem)
    pltpu.make_async_copy(tbl3.at[pl.ds(0, _RG)], rows_v, sem).wait()

    lanes = lax.iota(jnp.int32, _L)

    @plsc.parallel_loop(0, _BPW, unroll=4)
    def elem(b):
        p = _NE * b
        acc = None
        for k in range(_D // _L):
            s = pl.ds(k * _L, _L)
            t = rows_v[lax.shift_right_logical(p, 3),
                       lax.bitwise_and(p, 7), s] \
                * rows_v[lax.shift_right_logical(p + 1, 3),
                         lax.bitwise_and(p + 1, 7), s] \
                * rows_v[lax.shift_right_logical(p + 2, 3),
                         lax.bitwise_and(p + 2, 7), s]
            acc = t if acc is None else acc + t
        plsc.store_scatter(q, [lanes, jnp.full((_L,), b, jnp.int32)], acc)

    for g in range(_BPW // _L):
        s = pl.ds(g * _L, _L)
        tot = q[0, s]
        for r in range(1, _L):
            tot = tot + q[r, s]
        ov[s] = 1.0 / (1.0 + jnp.exp(-tot))
    pltpu.sync_copy(ov, out.at[pl.ds(base, _BPW)])


@functools.partial(jax.jit, static_argnames=())
def _run(xf, tbl3):
    mesh = plsc.VectorSubcoreMesh(
        core_axis_name="c", subcore_axis_name="s",
        num_cores=_NC, num_subcores=_NS,
    )
    return pl.kernel(
        _body,
        out_type=jax.ShapeDtypeStruct((_B,), jnp.float32),
        mesh=mesh,
        compiler_params=pltpu.CompilerParams(
            needs_layout_passes=False, use_tc_tiling_on_sc=True),
        scratch_types=[
            pltpu.VMEM((_IPW,), jnp.int32),
            pltpu.VMEM((_RG, 8, _D), jnp.float32),
            pltpu.VMEM((_L, _BPW), jnp.float32),
            pltpu.VMEM((_BPW,), jnp.float32),
            pltpu.SemaphoreType.DMA,
        ],
    )(xf, tbl3)


def kernel(x, node_embedding):
    xf = x.astype(jnp.int32).reshape(-1)
    v = node_embedding.shape[0]
    tbl3 = node_embedding.reshape(v // 8, 8, _D)
    return _run(xf, tbl3)


# speedup vs baseline: 1.0091x; 1.0091x over previous
import functools

import jax
import jax.numpy as jnp
from jax import lax
from jax.experimental import pallas as pl
from jax.experimental.pallas import tpu as pltpu
from jax.experimental.pallas import tpu_sc as plsc

_B = 4096
_D = 64
_NE = 3
_NC, _NS = 2, 16
_NW = _NC * _NS
_BPW = _B // _NW
_L = 16
_IPW = _BPW * _NE
_RG = _IPW // 8


def _body(xf, tbl3, out, iv, rows_v, q, ov, sem, sem2):
    wid = lax.axis_index("s") * _NC + lax.axis_index("c")
    base = wid * _BPW
    pltpu.sync_copy(xf.at[pl.ds(wid * _IPW, _IPW)], iv)

    _HG = _IPW // _L // 2

    def fire_half(h, hsem):
        @plsc.parallel_loop(h * _HG, (h + 1) * _HG)
        def fire(g):
            vec = iv[pl.ds(g * _L, _L)]
            tvec = lax.shift_right_logical(vec, 3)
            rvec = lax.bitwise_and(vec, 7)
            for e in range(_L):
                pltpu.async_copy(
                    tbl3.at[tvec[e], rvec[e]],
                    rows_v.at[2 * g + e // 8, e % 8],
                    hsem)

    lanes = lax.iota(jnp.int32, _L)

    def compute_half(h):
        @plsc.parallel_loop(h * _BPW // 2, (h + 1) * _BPW // 2, unroll=2)
        def elem(b):
            p = _NE * b
            acc = None
            for k in range(_D // _L):
                s = pl.ds(k * _L, _L)
                t = rows_v[lax.shift_right_logical(p, 3),
                           lax.bitwise_and(p, 7), s] \
                    * rows_v[lax.shift_right_logical(p + 1, 3),
                             lax.bitwise_and(p + 1, 7), s] \
                    * rows_v[lax.shift_right_logical(p + 2, 3),
                             lax.bitwise_and(p + 2, 7), s]
                acc = t if acc is None else acc + t
            plsc.store_scatter(q, [lanes, jnp.full((_L,), b, jnp.int32)], acc)

    fire_half(0, sem)
    fire_half(1, sem2)
    pltpu.make_async_copy(tbl3.at[pl.ds(0, _RG // 2)],
                          rows_v.at[pl.ds(0, _RG // 2)], sem).wait()
    compute_half(0)
    pltpu.make_async_copy(tbl3.at[pl.ds(0, _RG // 2)],
                          rows_v.at[pl.ds(_RG // 2, _RG // 2)], sem2).wait()
    compute_half(1)

    for g in range(_BPW // _L):
        s = pl.ds(g * _L, _L)
        tot = q[0, s]
        for r in range(1, _L):
            tot = tot + q[r, s]
        ov[s] = 1.0 / (1.0 + jnp.exp(-tot))
    pltpu.sync_copy(ov, out.at[pl.ds(base, _BPW)])


@functools.partial(jax.jit, static_argnames=())
def _run(xf, tbl3):
    mesh = plsc.VectorSubcoreMesh(
        core_axis_name="c", subcore_axis_name="s",
        num_cores=_NC, num_subcores=_NS,
    )
    return pl.kernel(
        _body,
        out_type=jax.ShapeDtypeStruct((_B,), jnp.float32),
        mesh=mesh,
        compiler_params=pltpu.CompilerParams(
            needs_layout_passes=False, use_tc_tiling_on_sc=True),
        scratch_types=[
            pltpu.VMEM((_IPW,), jnp.int32),
            pltpu.VMEM((_RG, 8, _D), jnp.float32),
            pltpu.VMEM((_L, _BPW), jnp.float32),
            pltpu.VMEM((_BPW,), jnp.float32),
            pltpu.SemaphoreType.DMA,
            pltpu.SemaphoreType.DMA,
        ],
    )(xf, tbl3)


def kernel(x, node_embedding):
    xf = x.astype(jnp.int32).reshape(-1)
    v = node_embedding.shape[0]
    tbl3 = node_embedding.reshape(v // 8, 8, _D)
    return _run(xf, tbl3)


# speedup vs baseline: 1.0141x; 1.0050x over previous
import functools

import jax
import jax.numpy as jnp
from jax import lax
from jax.experimental import pallas as pl
from jax.experimental.pallas import tpu as pltpu
from jax.experimental.pallas import tpu_sc as plsc

_B = 4096
_D = 64
_NE = 3
_NC, _NS = 2, 16
_NW = _NC * _NS
_BPW = _B // _NW
_L = 16
_IPW = _BPW * _NE
_RG = _IPW // 8


def _body(xf, tbl3, out, iv, rows_v, q, ov, sem):
    wid = lax.axis_index("s") * _NC + lax.axis_index("c")
    base = wid * _BPW
    pltpu.sync_copy(xf.at[pl.ds(wid * _IPW, _IPW)], iv)

    @plsc.parallel_loop(0, _IPW // _L)
    def fire(g):
        vec = iv[pl.ds(g * _L, _L)]
        tvec = lax.shift_right_logical(vec, 3)
        rvec = lax.bitwise_and(vec, 7)
        for e in range(_L):
            pltpu.async_copy(
                tbl3.at[tvec[e], rvec[e]],
                rows_v.at[2 * g + e // 8, e % 8],
                sem)
    pltpu.make_async_copy(tbl3.at[pl.ds(0, _RG)], rows_v, sem).wait()

    lanes = lax.iota(jnp.int32, _L)

    @plsc.parallel_loop(0, _BPW, unroll=2)
    def elem(b):
        p = _NE * b
        acc = None
        for k in range(_D // _L):
            s = pl.ds(k * _L, _L)
            t = rows_v[lax.shift_right_logical(p, 3),
                       lax.bitwise_and(p, 7), s] \
                * rows_v[lax.shift_right_logical(p + 1, 3),
                         lax.bitwise_and(p + 1, 7), s] \
                * rows_v[lax.shift_right_logical(p + 2, 3),
                         lax.bitwise_and(p + 2, 7), s]
            acc = t if acc is None else acc + t
        plsc.store_scatter(q, [lanes, jnp.full((_L,), b, jnp.int32)], acc)

    for g in range(_BPW // _L):
        s = pl.ds(g * _L, _L)
        tot = q[0, s]
        for r in range(1, _L):
            tot = tot + q[r, s]
        ov[s] = 1.0 / (1.0 + jnp.exp(-tot))
    pltpu.sync_copy(ov, out.at[pl.ds(base, _BPW)])


@functools.partial(jax.jit, static_argnames=())
def _run(xf, tbl3):
    mesh = plsc.VectorSubcoreMesh(
        core_axis_name="c", subcore_axis_name="s",
        num_cores=_NC, num_subcores=_NS,
    )
    return pl.kernel(
        _body,
        out_type=jax.ShapeDtypeStruct((_B,), jnp.float32),
        mesh=mesh,
        compiler_params=pltpu.CompilerParams(
            needs_layout_passes=False, use_tc_tiling_on_sc=True),
        scratch_types=[
            pltpu.VMEM((_IPW,), jnp.int32),
            pltpu.VMEM((_RG, 8, _D), jnp.float32),
            pltpu.VMEM((_L, _BPW), jnp.float32),
            pltpu.VMEM((_BPW,), jnp.float32),
            pltpu.SemaphoreType.DMA,
        ],
    )(xf, tbl3)


def kernel(x, node_embedding):
    xf = x.astype(jnp.int32).reshape(-1)
    v = node_embedding.shape[0]
    tbl3 = node_embedding.reshape(v // 8, 8, _D)
    return _run(xf, tbl3)
